# baseline (speedup 1.0000x reference)
import functools

import jax
import jax.numpy as jnp
from jax import lax
from jax.experimental import pallas as pl
from jax.experimental.pallas import tpu as pltpu
from jax.experimental.pallas import tpu_sc as plsc

EMBED_DIM = 32
ROW_W = 128
RPG = ROW_W // EMBED_DIM
CHUNK = 128

_info = plsc.get_sparse_core_info()
_NC, _NS = _info.num_cores, _info.num_subcores
_NW = _NC * _NS


@functools.partial(jax.jit, static_argnums=(3, 4))
def _gather(table4, idx, aux, n_fields, b_per_w):
    batch = idx.shape[0] // n_fields
    rows_per_w = b_per_w * n_fields
    n_chunks = rows_per_w // CHUNK
    mesh = plsc.VectorSubcoreMesh(core_axis_name="c", subcore_axis_name="s")

    @functools.partial(
        pl.kernel,
        mesh=mesh,
        compiler_params=pltpu.CompilerParams(needs_layout_passes=False),
        out_type=jax.ShapeDtypeStruct((n_fields, EMBED_DIM, batch), jnp.float32),
        scratch_types=[
            pltpu.VMEM((rows_per_w,), jnp.int32),
            pltpu.VMEM((rows_per_w,), jnp.int32),
            pltpu.VMEM((CHUNK, ROW_W), jnp.float32),
            pltpu.VMEM((n_fields, EMBED_DIM, b_per_w), jnp.float32),
            pltpu.SemaphoreType.DMA,
        ],
    )
    def k(table_hbm, idx_hbm, aux_hbm, out_hbm, idx_v, aux_v, rows_v, o_v, gsem):
        wid = lax.axis_index("s") * _NC + lax.axis_index("c")
        base = wid * rows_per_w
        b0 = wid * b_per_w
        pltpu.sync_copy(idx_hbm.at[pl.ds(base, rows_per_w)], idx_v)
        pltpu.sync_copy(aux_hbm, aux_v)

        def chunk_body(c, carry):
            for s in range(CHUNK // 16):
                qi = idx_v[pl.ds(c * CHUNK + s * 16, 16)] >> 2
                pltpu.async_copy(
                    table_hbm.at[qi], rows_v.at[pl.ds(s * 16, 16)], gsem
                )
            for s in range(CHUNK // 16):
                qi = idx_v[pl.ds(c * CHUNK + s * 16, 16)] >> 2
                pltpu.make_async_copy(
                    table_hbm.at[qi], rows_v.at[pl.ds(s * 16, 16)], gsem
                ).wait()

            def select(rc, carry2):
                off = c * CHUNK + rc * 16
                m = idx_v[pl.ds(off, 16)] & 3
                aux = aux_v[pl.ds(off, 16)]
                f = aux & 31
                bp = (aux >> 5) & 127
                r = aux >> 12
                col0 = m * EMBED_DIM
                for e in range(EMBED_DIM):
                    vals = plsc.load_gather(rows_v, [r, col0 + e])
                    plsc.store_scatter(
                        o_v, [f, jnp.full((16,), e, jnp.int32), bp], vals
                    )
                return carry2

            lax.fori_loop(0, CHUNK // 16, select, 0)
            return carry

        lax.fori_loop(0, n_chunks, chunk_body, 0)
        pltpu.sync_copy(o_v, out_hbm.at[:, :, pl.ds(b0, b_per_w)])

    return k(table4, idx, aux)


def kernel(spatialgrid, comparison_grid):
    batch, n_fields = comparison_grid.shape[0], comparison_grid.shape[1]
    b_per_w = batch // _NW
    idx = comparison_grid.reshape(batch * n_fields)
    table4 = spatialgrid.reshape(spatialgrid.shape[0] // RPG, ROW_W)
    j = jnp.arange(b_per_w * n_fields, dtype=jnp.int32)
    aux = (j % n_fields) | ((j // n_fields) << 5) | ((j % CHUNK) << 12)
    o = _gather(table4, idx, aux, n_fields, b_per_w)
    return jnp.transpose(o, (2, 0, 1))
